# baseline (speedup 1.0000x reference)
import functools

import jax
import jax.numpy as jnp
from jax import lax
from jax.experimental import pallas as pl
from jax.experimental.pallas import tpu as pltpu
from jax.experimental.pallas import tpu_sc as plsc

N = 10000
E = 320000
H = 128
HH = H // 2
INV = float(1.0 / (1.0 + 1e-5) ** 0.5)

NC = 2
NS = 16
NW = NC * NS
EPW = E // NW
BC = 40
NBLK = EPW // BC
NPAD = 10240
RPT = NPAD // NS

BE = 2000
BN = 1000

_mesh = plsc.VectorSubcoreMesh(core_axis_name="c", subcore_axis_name="s")


def _edge_dense_body(erp_ref, m0_ref, v0_ref, g0_ref, be0_ref, cw1_ref,
                     m1_ref, w1_ref, sig_ref, ce_ref):
    er = erp_ref[...]
    z = jnp.dot(er, m0_ref[...], preferred_element_type=jnp.float32) + v0_ref[...]
    sig_ref[...] = 1.0 / (1.0 + jnp.exp(-z))
    t = jnp.maximum(g0_ref[...] * z + be0_ref[...], 0.0)
    ce_ref[...] = (jnp.dot(t, cw1_ref[...], preferred_element_type=jnp.float32)
                   + jnp.dot(er, m1_ref[...], preferred_element_type=jnp.float32)
                   + w1_ref[...])


def _edge_dense(erp, m0, v0, g0, be0, cw1, m1, w1):
    nblk = E // BE
    full = lambda shape: pl.BlockSpec(shape, lambda i: (0, 0))
    return pl.pallas_call(
        _edge_dense_body,
        grid=(nblk,),
        in_specs=[
            pl.BlockSpec((BE, 8), lambda i: (i, 0)),
            full((8, H)), full((1, H)), full((1, H)), full((1, H)),
            full((H, H)), full((8, H)), full((1, H)),
        ],
        out_specs=[
            pl.BlockSpec((BE, H), lambda i: (i, 0)),
            pl.BlockSpec((BE, H), lambda i: (i, 0)),
        ],
        out_shape=[
            jax.ShapeDtypeStruct((E, H), jnp.float32),
            jax.ShapeDtypeStruct((E, H), jnp.float32),
        ],
    )(erp, m0, v0, g0, be0, cw1, m1, w1)


def _node_dense_body(dp_ref, sv_ref, a0_ref, b0_ref, g0_ref, bx0_ref,
                     aw_ref, ab_ref, bw_ref, bb_ref, dw_ref, db_ref,
                     ew_ref, eb_ref,
                     x1_ref, ax_ref, dx_ref, eba_ref, ebb_ref):
    den = dp_ref[0] + dp_ref[1]
    r = den / (den + 1e-6)
    x1 = sv_ref[...] + jnp.maximum(
        g0_ref[...] * (a0_ref[...] + b0_ref[...] * r) + bx0_ref[...], 0.0)
    x1_ref[...] = x1
    ax_ref[...] = jnp.dot(x1, aw_ref[...], preferred_element_type=jnp.float32) + ab_ref[...]
    dx_ref[...] = jnp.dot(x1, dw_ref[...], preferred_element_type=jnp.float32) + db_ref[...]
    ex = jnp.dot(x1, ew_ref[...], preferred_element_type=jnp.float32) + eb_ref[...]
    b = jnp.dot(x1, bw_ref[...], preferred_element_type=jnp.float32) + bb_ref[...]
    eba_ref[...] = jnp.concatenate([ex[:, :HH], b[:, :HH]], axis=-1)
    ebb_ref[...] = jnp.concatenate([ex[:, HH:], b[:, HH:]], axis=-1)


def _node_dense(dparts, svec, a0, b0, g0, bx0, aw, ab, bw, bb, dw, db, ew, eb):
    nblk = N // BN
    full = lambda shape: pl.BlockSpec(shape, lambda i: (0, 0))
    outs = [jax.ShapeDtypeStruct((N, H), jnp.float32),
            jax.ShapeDtypeStruct((N, H), jnp.float32),
            jax.ShapeDtypeStruct((NPAD, H), jnp.float32),
            jax.ShapeDtypeStruct((NPAD, H), jnp.float32),
            jax.ShapeDtypeStruct((NPAD, H), jnp.float32)]
    return pl.pallas_call(
        _node_dense_body,
        grid=(nblk,),
        in_specs=[
            pl.BlockSpec((2, BN, H), lambda i: (0, i, 0)),
            full((1, H)), full((1, H)), full((1, H)), full((1, H)), full((1, H)),
            full((H, H)), full((1, H)), full((H, H)), full((1, H)),
            full((H, H)), full((1, H)), full((H, H)), full((1, H)),
        ],
        out_specs=[pl.BlockSpec((BN, H), lambda i: (i, 0))] * 5,
        out_shape=outs,
    )(dparts, svec, a0, b0, g0, bx0, aw, ab, bw, bb, dw, db, ew, eb)


def _final_body(x1_ref, ax_ref, nda_ref, ndb_ref, g1_ref,
                bx1_ref, out_ref):
    num = jnp.concatenate([nda_ref[0, :, :HH] + nda_ref[1, :, :HH],
                           ndb_ref[0, :, :HH] + ndb_ref[1, :, :HH]], axis=-1)
    den = jnp.concatenate([nda_ref[0, :, HH:] + nda_ref[1, :, HH:],
                           ndb_ref[0, :, HH:] + ndb_ref[1, :, HH:]], axis=-1)
    out_ref[...] = x1_ref[...] + jnp.maximum(
        g1_ref[...] * (ax_ref[...] + num / (den + 1e-6)) + bx1_ref[...], 0.0)


def _final(x1, ax, nda, ndb, g1, bx1):
    nblk = N // BN
    full = lambda shape: pl.BlockSpec(shape, lambda i: (0, 0))
    comb = pl.BlockSpec((2, BN, H), lambda i: (0, i, 0))
    return pl.pallas_call(
        _final_body,
        grid=(nblk,),
        in_specs=[
            pl.BlockSpec((BN, H), lambda i: (i, 0)),
            pl.BlockSpec((BN, H), lambda i: (i, 0)),
            comb, comb,
            full((1, H)), full((1, H)),
        ],
        out_specs=pl.BlockSpec((BN, H), lambda i: (i, 0)),
        out_shape=jax.ShapeDtypeStruct((N, H), jnp.float32),
    )(x1, ax, nda, ndb, g1, bx1)


@functools.partial(
    pl.kernel,
    out_type=jax.ShapeDtypeStruct((NC, NPAD, H), jnp.float32),
    mesh=_mesh,
    scratch_types=[
        pltpu.MemorySpace.VMEM((BC,), jnp.int32),
        pltpu.MemorySpace.VMEM((2, BC, H), jnp.float32),
        pltpu.MemorySpace.VMEM_SHARED((NPAD, H), jnp.float32),
        pltpu.SemaphoreType.DMA,
        pltpu.SemaphoreType.DMA,
    ],
)
def _den0_sc(dst_hbm, sig_hbm, zero_hbm, out_hbm, idx_v, rows2, acc, sem0, sem1):
    cid = lax.axis_index("c")
    sid = lax.axis_index("s")
    wid = sid * NC + cid
    r0 = pl.multiple_of(sid * RPT, 8)
    pltpu.sync_copy(zero_hbm.at[pl.ds(r0, RPT)], acc.at[pl.ds(r0, RPT)])
    plsc.subcore_barrier()

    e0 = pl.multiple_of(wid * EPW, 8)
    pltpu.async_copy(sig_hbm.at[pl.ds(e0, BC)], rows2.at[0], sem0)

    def body(j, carry):
        p = j % 2
        base = pl.multiple_of(wid * EPW + j * BC, 8)
        nb = pl.multiple_of(wid * EPW + (j + 1) * BC, 8)
        for pp, sem, nsem in ((0, sem0, sem1), (1, sem1, sem0)):
            @pl.when(p == pp)
            def _stage():
                pltpu.make_async_copy(sig_hbm.at[pl.ds(0, BC)], rows2.at[pp], sem).wait()

                @pl.when(j + 1 < NBLK)
                def _prefetch():
                    pltpu.async_copy(sig_hbm.at[pl.ds(nb, BC)], rows2.at[1 - pp], nsem)

        pltpu.sync_copy(dst_hbm.at[pl.ds(base, BC)], idx_v)
        pltpu.sync_copy(rows2.at[p], acc.at[idx_v], add=True)
        return carry

    lax.fori_loop(0, NBLK, body, 0)
    plsc.subcore_barrier()
    pltpu.sync_copy(acc.at[pl.ds(r0, RPT)], out_hbm.at[cid, pl.ds(r0, RPT)])


def _make_agg(h):
    @functools.partial(
        pl.kernel,
        out_type=jax.ShapeDtypeStruct((NC, NPAD, H), jnp.float32),
        mesh=_mesh,
        scratch_types=[
            pltpu.MemorySpace.VMEM((2, BC), jnp.int32),
            pltpu.MemorySpace.VMEM((2, BC), jnp.int32),
            pltpu.MemorySpace.VMEM((BC,), jnp.int32),
            pltpu.MemorySpace.VMEM((2, BC, H), jnp.float32),
            pltpu.MemorySpace.VMEM((2, BC, H), jnp.float32),
            pltpu.MemorySpace.VMEM((2, BC, H), jnp.float32),
            pltpu.MemorySpace.VMEM((BC, H), jnp.float32),
            pltpu.MemorySpace.VMEM_SHARED((NPAD, H), jnp.float32),
            pltpu.SemaphoreType.DMA,
            pltpu.SemaphoreType.DMA,
        ],
    )
    def _agg_sc(src_hbm, dst_hbm, ce_hbm, d_hbm, eb_hbm, zero_hbm, nd_hbm,
                idxg_s, idxg_d, idx_scat, drows2, ebrows2, cerows2, psbuf,
                acc, sem0, sem1):
        cid = lax.axis_index("c")
        sid = lax.axis_index("s")
        wid = sid * NC + cid
        r0 = pl.multiple_of(sid * RPT, 8)
        pltpu.sync_copy(zero_hbm.at[pl.ds(r0, RPT)], acc.at[pl.ds(r0, RPT)])
        plsc.subcore_barrier()

        e0 = pl.multiple_of(wid * EPW, 8)
        pltpu.sync_copy(src_hbm.at[pl.ds(e0, BC)], idxg_s.at[0])
        pltpu.sync_copy(dst_hbm.at[pl.ds(e0, BC)], idxg_d.at[0])
        pltpu.async_copy(d_hbm.at[idxg_d.at[0]], drows2.at[0], sem0)
        pltpu.async_copy(eb_hbm.at[idxg_s.at[0]], ebrows2.at[0], sem0)
        pltpu.async_copy(ce_hbm.at[pl.ds(e0, BC)], cerows2.at[0], sem0)

        def body(j, carry):
            p = j % 2
            base = pl.multiple_of(wid * EPW + j * BC, 8)
            nb = pl.multiple_of(wid * EPW + (j + 1) * BC, 8)
            for pp, sem, nsem in ((0, sem0, sem1), (1, sem1, sem0)):
                @pl.when(p == pp)
                def _stage():
                    pltpu.make_async_copy(d_hbm.at[pl.ds(0, BC)], drows2.at[pp], sem).wait()
                    pltpu.make_async_copy(eb_hbm.at[pl.ds(0, BC)], ebrows2.at[pp], sem).wait()
                    pltpu.make_async_copy(ce_hbm.at[pl.ds(0, BC)], cerows2.at[pp], sem).wait()

                    @pl.when(j + 1 < NBLK)
                    def _prefetch():
                        pltpu.sync_copy(src_hbm.at[pl.ds(nb, BC)], idxg_s.at[1 - pp])
                        pltpu.sync_copy(dst_hbm.at[pl.ds(nb, BC)], idxg_d.at[1 - pp])
                        pltpu.async_copy(d_hbm.at[idxg_d.at[1 - pp]], drows2.at[1 - pp], nsem)
                        pltpu.async_copy(eb_hbm.at[idxg_s.at[1 - pp]], ebrows2.at[1 - pp], nsem)
                        pltpu.async_copy(ce_hbm.at[pl.ds(nb, BC)], cerows2.at[1 - pp], nsem)

            def edge(i, ecarry):
                for c in range(HH // 16):
                    hsl = pl.ds(h * HH + c * 16, 16)
                    z = (drows2[p, i, hsl] + ebrows2[p, i, pl.ds(c * 16, 16)]
                         + cerows2[p, i, hsl])
                    sg = 1.0 / (1.0 + jnp.exp(-z))
                    psbuf[i, pl.ds(c * 16, 16)] = sg * ebrows2[p, i, pl.ds(HH + c * 16, 16)]
                    psbuf[i, pl.ds(HH + c * 16, 16)] = sg
                return ecarry

            lax.fori_loop(0, BC, edge, 0)
            pltpu.sync_copy(dst_hbm.at[pl.ds(base, BC)], idx_scat)
            pltpu.sync_copy(psbuf, acc.at[idx_scat], add=True)
            return carry

        lax.fori_loop(0, NBLK, body, 0)
        plsc.subcore_barrier()
        pltpu.sync_copy(acc.at[pl.ds(r0, RPT)], nd_hbm.at[cid, pl.ds(r0, RPT)])

    return _agg_sc


_agg_a = _make_agg(0)
_agg_b = _make_agg(1)


def kernel(edge_index_old, edge_attr_old, flow_old, num_nodes, edge_proj_w,
           edge_proj_b, A_w, A_b, B_w, B_b, C_w, C_b, D_w, D_b, E_w, E_b,
           bn_x_g, bn_x_b, bn_e_g, bn_e_b):
    src = edge_index_old[0].astype(jnp.int32)
    dst = edge_index_old[1].astype(jnp.int32)
    zero = (jnp.asarray(num_nodes) - N).astype(jnp.float32)
    s = 1.0 + zero

    erp = jnp.concatenate(
        [edge_attr_old, flow_old, jnp.zeros((E, 4), jnp.float32)], axis=-1)
    wp8 = jnp.concatenate([edge_proj_w, jnp.zeros((4, H), jnp.float32)], axis=0)

    a0 = s * A_w[0].sum(0) + A_b[0]
    b0 = s * B_w[0].sum(0) + B_b[0]
    d0 = s * D_w[0].sum(0) + D_b[0]
    ex0 = s * E_w[0].sum(0) + E_b[0]
    m0 = wp8 @ C_w[0]
    v0 = edge_proj_b @ C_w[0] + C_b[0] + d0 + ex0
    m1 = wp8 @ C_w[1]
    w1 = edge_proj_b @ C_w[1] + C_b[1]

    g0 = (bn_e_g[0] * INV)[None]
    be0 = bn_e_b[0][None]
    gx0 = (bn_x_g[0] * INV)[None]
    bx0 = bn_x_b[0][None]
    gx1 = (bn_x_g[1] * INV)[None]
    bx1 = bn_x_b[1][None]
    svec = jnp.broadcast_to(s, (1, H)).astype(jnp.float32)
    zeros128 = jnp.zeros((NPAD, H), jnp.float32)

    sig0, ce1 = _edge_dense(erp, m0, v0[None], g0, be0, C_w[1], m1, w1[None])
    dparts = _den0_sc(dst, sig0, zeros128)
    x1, ax, dx, eba, ebb = _node_dense(
        dparts, svec, a0[None], b0[None], gx0, bx0,
        A_w[1], A_b[1][None], B_w[1], B_b[1][None],
        D_w[1], D_b[1][None], E_w[1], E_b[1][None])
    nda = _agg_a(src, dst, ce1, dx, eba, zeros128)
    ndb = _agg_b(src, dst, ce1, dx, ebb, zeros128)
    return _final(x1, ax, nda, ndb, gx1, bx1)


# speedup vs baseline: 1.8990x; 1.8990x over previous
import functools

import jax
import jax.numpy as jnp
from jax import lax
from jax.experimental import pallas as pl
from jax.experimental.pallas import tpu as pltpu
from jax.experimental.pallas import tpu_sc as plsc

N = 10000
E = 320000
H = 128
HH = H // 2
INV = float(1.0 / (1.0 + 1e-5) ** 0.5)

NC = 2
NS = 16
NW = NC * NS
EPW = E // NW
BC = 40
NBLK = EPW // BC
NPAD = 10240
RPT = NPAD // NS

BE = 2000
BN = 1000

_mesh = plsc.VectorSubcoreMesh(core_axis_name="c", subcore_axis_name="s")


def _edge_dense_body(erp_ref, m0_ref, v0_ref, g0_ref, be0_ref, cw1_ref,
                     m1_ref, w1_ref, sig_ref, ce_ref):
    er = erp_ref[...]
    z = jnp.dot(er, m0_ref[...], preferred_element_type=jnp.float32) + v0_ref[...]
    sig_ref[...] = 1.0 / (1.0 + jnp.exp(-z))
    t = jnp.maximum(g0_ref[...] * z + be0_ref[...], 0.0)
    ce_ref[...] = (jnp.dot(t, cw1_ref[...], preferred_element_type=jnp.float32)
                   + jnp.dot(er, m1_ref[...], preferred_element_type=jnp.float32)
                   + w1_ref[...])


def _edge_dense(erp, m0, v0, g0, be0, cw1, m1, w1):
    nblk = E // BE
    full = lambda shape: pl.BlockSpec(shape, lambda i: (0, 0))
    return pl.pallas_call(
        _edge_dense_body,
        grid=(nblk,),
        in_specs=[
            pl.BlockSpec((BE, 8), lambda i: (i, 0)),
            full((8, H)), full((1, H)), full((1, H)), full((1, H)),
            full((H, H)), full((8, H)), full((1, H)),
        ],
        out_specs=[
            pl.BlockSpec((BE, H), lambda i: (i, 0)),
            pl.BlockSpec((BE, H), lambda i: (i, 0)),
        ],
        out_shape=[
            jax.ShapeDtypeStruct((E, H), jnp.float32),
            jax.ShapeDtypeStruct((E, H), jnp.float32),
        ],
    )(erp, m0, v0, g0, be0, cw1, m1, w1)


def _node_dense_body(dp_ref, sv_ref, a0_ref, b0_ref, g0_ref, bx0_ref,
                     aw_ref, ab_ref, bw_ref, bb_ref, dw_ref, db_ref,
                     ew_ref, eb_ref,
                     x1_ref, ax_ref, dx_ref, eba_ref, ebb_ref):
    den = dp_ref[0] + dp_ref[1]
    r = den / (den + 1e-6)
    x1 = sv_ref[...] + jnp.maximum(
        g0_ref[...] * (a0_ref[...] + b0_ref[...] * r) + bx0_ref[...], 0.0)
    x1_ref[...] = x1
    ax_ref[...] = jnp.dot(x1, aw_ref[...], preferred_element_type=jnp.float32) + ab_ref[...]
    dx_ref[...] = jnp.dot(x1, dw_ref[...], preferred_element_type=jnp.float32) + db_ref[...]
    ex = jnp.dot(x1, ew_ref[...], preferred_element_type=jnp.float32) + eb_ref[...]
    b = jnp.dot(x1, bw_ref[...], preferred_element_type=jnp.float32) + bb_ref[...]
    eba_ref[...] = jnp.concatenate([ex[:, :HH], b[:, :HH]], axis=-1)
    ebb_ref[...] = jnp.concatenate([ex[:, HH:], b[:, HH:]], axis=-1)


def _node_dense(dparts, svec, a0, b0, g0, bx0, aw, ab, bw, bb, dw, db, ew, eb):
    nblk = N // BN
    full = lambda shape: pl.BlockSpec(shape, lambda i: (0, 0))
    outs = [jax.ShapeDtypeStruct((N, H), jnp.float32),
            jax.ShapeDtypeStruct((N, H), jnp.float32),
            jax.ShapeDtypeStruct((NPAD, H), jnp.float32),
            jax.ShapeDtypeStruct((NPAD, H), jnp.float32),
            jax.ShapeDtypeStruct((NPAD, H), jnp.float32)]
    return pl.pallas_call(
        _node_dense_body,
        grid=(nblk,),
        in_specs=[
            pl.BlockSpec((2, BN, H), lambda i: (0, i, 0)),
            full((1, H)), full((1, H)), full((1, H)), full((1, H)), full((1, H)),
            full((H, H)), full((1, H)), full((H, H)), full((1, H)),
            full((H, H)), full((1, H)), full((H, H)), full((1, H)),
        ],
        out_specs=[pl.BlockSpec((BN, H), lambda i: (i, 0))] * 5,
        out_shape=outs,
    )(dparts, svec, a0, b0, g0, bx0, aw, ab, bw, bb, dw, db, ew, eb)


def _final_body(x1_ref, ax_ref, nda_ref, ndb_ref, g1_ref,
                bx1_ref, out_ref):
    num = jnp.concatenate([nda_ref[0, :, :HH] + nda_ref[1, :, :HH],
                           ndb_ref[0, :, :HH] + ndb_ref[1, :, :HH]], axis=-1)
    den = jnp.concatenate([nda_ref[0, :, HH:] + nda_ref[1, :, HH:],
                           ndb_ref[0, :, HH:] + ndb_ref[1, :, HH:]], axis=-1)
    out_ref[...] = x1_ref[...] + jnp.maximum(
        g1_ref[...] * (ax_ref[...] + num / (den + 1e-6)) + bx1_ref[...], 0.0)


def _final(x1, ax, nda, ndb, g1, bx1):
    nblk = N // BN
    full = lambda shape: pl.BlockSpec(shape, lambda i: (0, 0))
    comb = pl.BlockSpec((2, BN, H), lambda i: (0, i, 0))
    return pl.pallas_call(
        _final_body,
        grid=(nblk,),
        in_specs=[
            pl.BlockSpec((BN, H), lambda i: (i, 0)),
            pl.BlockSpec((BN, H), lambda i: (i, 0)),
            comb, comb,
            full((1, H)), full((1, H)),
        ],
        out_specs=pl.BlockSpec((BN, H), lambda i: (i, 0)),
        out_shape=jax.ShapeDtypeStruct((N, H), jnp.float32),
    )(x1, ax, nda, ndb, g1, bx1)


@functools.partial(
    pl.kernel,
    out_type=jax.ShapeDtypeStruct((NC, NPAD, H), jnp.float32),
    mesh=_mesh,
    scratch_types=[
        pltpu.MemorySpace.VMEM((BC,), jnp.int32),
        pltpu.MemorySpace.VMEM((2, BC, H), jnp.float32),
        pltpu.MemorySpace.VMEM_SHARED((NPAD, H), jnp.float32),
        pltpu.SemaphoreType.DMA,
        pltpu.SemaphoreType.DMA,
    ],
)
def _den0_sc(dst_hbm, sig_hbm, zero_hbm, out_hbm, idx_v, rows2, acc, sem0, sem1):
    cid = lax.axis_index("c")
    sid = lax.axis_index("s")
    wid = sid * NC + cid
    r0 = pl.multiple_of(sid * RPT, 8)
    pltpu.sync_copy(zero_hbm.at[pl.ds(r0, RPT)], acc.at[pl.ds(r0, RPT)])
    plsc.subcore_barrier()

    e0 = pl.multiple_of(wid * EPW, 8)
    pltpu.async_copy(sig_hbm.at[pl.ds(e0, BC)], rows2.at[0], sem0)

    def body(j, carry):
        p = j % 2
        base = pl.multiple_of(wid * EPW + j * BC, 8)
        nb = pl.multiple_of(wid * EPW + (j + 1) * BC, 8)
        for pp, sem, nsem in ((0, sem0, sem1), (1, sem1, sem0)):
            @pl.when(p == pp)
            def _stage():
                pltpu.make_async_copy(sig_hbm.at[pl.ds(0, BC)], rows2.at[pp], sem).wait()

                @pl.when(j + 1 < NBLK)
                def _prefetch():
                    pltpu.async_copy(sig_hbm.at[pl.ds(nb, BC)], rows2.at[1 - pp], nsem)

        pltpu.sync_copy(dst_hbm.at[pl.ds(base, BC)], idx_v)
        pltpu.sync_copy(rows2.at[p], acc.at[idx_v], add=True)
        return carry

    lax.fori_loop(0, NBLK, body, 0)
    plsc.subcore_barrier()
    pltpu.sync_copy(acc.at[pl.ds(r0, RPT)], out_hbm.at[cid, pl.ds(r0, RPT)])


def _make_agg(h):
    @functools.partial(
        pl.kernel,
        out_type=jax.ShapeDtypeStruct((NC, NPAD, H), jnp.float32),
        mesh=_mesh,
        scratch_types=[
            pltpu.MemorySpace.VMEM((2, BC), jnp.int32),
            pltpu.MemorySpace.VMEM((2, BC), jnp.int32),
            pltpu.MemorySpace.VMEM((BC,), jnp.int32),
            pltpu.MemorySpace.VMEM((2, BC, H), jnp.float32),
            pltpu.MemorySpace.VMEM((2, BC, H), jnp.float32),
            pltpu.MemorySpace.VMEM((2, BC, H), jnp.float32),
            pltpu.MemorySpace.VMEM((BC, H), jnp.float32),
            pltpu.MemorySpace.VMEM_SHARED((NPAD, H), jnp.float32),
            pltpu.SemaphoreType.DMA,
            pltpu.SemaphoreType.DMA,
        ],
    )
    def _agg_sc(src_hbm, dst_hbm, ce_hbm, d_hbm, eb_hbm, zero_hbm, nd_hbm,
                idxg_s, idxg_d, idx_scat, drows2, ebrows2, cerows2, psbuf,
                acc, sem0, sem1):
        cid = lax.axis_index("c")
        sid = lax.axis_index("s")
        wid = sid * NC + cid
        r0 = pl.multiple_of(sid * RPT, 8)
        pltpu.sync_copy(zero_hbm.at[pl.ds(r0, RPT)], acc.at[pl.ds(r0, RPT)])
        plsc.subcore_barrier()

        e0 = pl.multiple_of(wid * EPW, 8)
        pltpu.sync_copy(src_hbm.at[pl.ds(e0, BC)], idxg_s.at[0])
        pltpu.sync_copy(dst_hbm.at[pl.ds(e0, BC)], idxg_d.at[0])
        pltpu.async_copy(d_hbm.at[idxg_d.at[0]], drows2.at[0], sem0)
        pltpu.async_copy(eb_hbm.at[idxg_s.at[0]], ebrows2.at[0], sem0)
        pltpu.async_copy(ce_hbm.at[pl.ds(e0, BC)], cerows2.at[0], sem0)

        def body(j, carry):
            p = j % 2
            base = pl.multiple_of(wid * EPW + j * BC, 8)
            nb = pl.multiple_of(wid * EPW + (j + 1) * BC, 8)
            for pp, sem, nsem in ((0, sem0, sem1), (1, sem1, sem0)):
                @pl.when(p == pp)
                def _stage():
                    pltpu.make_async_copy(d_hbm.at[pl.ds(0, BC)], drows2.at[pp], sem).wait()
                    pltpu.make_async_copy(eb_hbm.at[pl.ds(0, BC)], ebrows2.at[pp], sem).wait()
                    pltpu.make_async_copy(ce_hbm.at[pl.ds(0, BC)], cerows2.at[pp], sem).wait()

                    @pl.when(j + 1 < NBLK)
                    def _prefetch():
                        pltpu.sync_copy(src_hbm.at[pl.ds(nb, BC)], idxg_s.at[1 - pp])
                        pltpu.sync_copy(dst_hbm.at[pl.ds(nb, BC)], idxg_d.at[1 - pp])
                        pltpu.async_copy(d_hbm.at[idxg_d.at[1 - pp]], drows2.at[1 - pp], nsem)
                        pltpu.async_copy(eb_hbm.at[idxg_s.at[1 - pp]], ebrows2.at[1 - pp], nsem)
                        pltpu.async_copy(ce_hbm.at[pl.ds(nb, BC)], cerows2.at[1 - pp], nsem)

                    def edge(i, ecarry):
                        for c in range(HH // 16):
                            hsl = pl.ds(h * HH + c * 16, 16)
                            z = (drows2[pp, i, hsl] + ebrows2[pp, i, pl.ds(c * 16, 16)]
                                 + cerows2[pp, i, hsl])
                            sg = 1.0 / (1.0 + jnp.exp(-z))
                            psbuf[i, pl.ds(c * 16, 16)] = sg * ebrows2[pp, i, pl.ds(HH + c * 16, 16)]
                            psbuf[i, pl.ds(HH + c * 16, 16)] = sg
                        return ecarry

                    lax.fori_loop(0, BC, edge, 0)

            pltpu.sync_copy(dst_hbm.at[pl.ds(base, BC)], idx_scat)
            pltpu.sync_copy(psbuf, acc.at[idx_scat], add=True)
            return carry

        lax.fori_loop(0, NBLK, body, 0)
        plsc.subcore_barrier()
        pltpu.sync_copy(acc.at[pl.ds(r0, RPT)], nd_hbm.at[cid, pl.ds(r0, RPT)])

    return _agg_sc


_agg_a = _make_agg(0)
_agg_b = _make_agg(1)


def kernel(edge_index_old, edge_attr_old, flow_old, num_nodes, edge_proj_w,
           edge_proj_b, A_w, A_b, B_w, B_b, C_w, C_b, D_w, D_b, E_w, E_b,
           bn_x_g, bn_x_b, bn_e_g, bn_e_b):
    src = edge_index_old[0].astype(jnp.int32)
    dst = edge_index_old[1].astype(jnp.int32)
    zero = (jnp.asarray(num_nodes) - N).astype(jnp.float32)
    s = 1.0 + zero

    erp = jnp.concatenate(
        [edge_attr_old, flow_old, jnp.zeros((E, 4), jnp.float32)], axis=-1)
    wp8 = jnp.concatenate([edge_proj_w, jnp.zeros((4, H), jnp.float32)], axis=0)

    a0 = s * A_w[0].sum(0) + A_b[0]
    b0 = s * B_w[0].sum(0) + B_b[0]
    d0 = s * D_w[0].sum(0) + D_b[0]
    ex0 = s * E_w[0].sum(0) + E_b[0]
    m0 = wp8 @ C_w[0]
    v0 = edge_proj_b @ C_w[0] + C_b[0] + d0 + ex0
    m1 = wp8 @ C_w[1]
    w1 = edge_proj_b @ C_w[1] + C_b[1]

    g0 = (bn_e_g[0] * INV)[None]
    be0 = bn_e_b[0][None]
    gx0 = (bn_x_g[0] * INV)[None]
    bx0 = bn_x_b[0][None]
    gx1 = (bn_x_g[1] * INV)[None]
    bx1 = bn_x_b[1][None]
    svec = jnp.broadcast_to(s, (1, H)).astype(jnp.float32)
    zeros128 = jnp.zeros((NPAD, H), jnp.float32)

    sig0, ce1 = _edge_dense(erp, m0, v0[None], g0, be0, C_w[1], m1, w1[None])
    dparts = _den0_sc(dst, sig0, zeros128)
    x1, ax, dx, eba, ebb = _node_dense(
        dparts, svec, a0[None], b0[None], gx0, bx0,
        A_w[1], A_b[1][None], B_w[1], B_b[1][None],
        D_w[1], D_b[1][None], E_w[1], E_b[1][None])
    nda = _agg_a(src, dst, ce1, dx, eba, zeros128)
    ndb = _agg_b(src, dst, ce1, dx, ebb, zeros128)
    return _final(x1, ax, nda, ndb, gx1, bx1)
